# baseline (speedup 1.0000x reference)
import functools

import jax
import jax.numpy as jnp
from jax import lax
from jax.experimental import pallas as pl
from jax.experimental.pallas import tpu as pltpu
from jax.experimental.pallas import tpu_sc as plsc

_L = 16
_K = 512


def _scatter_chunks(n_points, n_chunks, pred_hbm, idx_hbm, pair,
                    acc_v, chunk0_v, chunk1_v, idx0_v, idx1_v, sem0, sem1):
    c_classes = chunk0_v.shape[0]
    bufs = ((chunk0_v, idx0_v, sem0), (chunk1_v, idx1_v, sem1))

    def start(i, slot):
        chunk_v, idx_v, sem = bufs[slot]
        base = i * _K
        pltpu.async_copy(pred_hbm.at[pair, :, pl.ds(base, _K)], chunk_v, sem)
        pltpu.async_copy(idx_hbm.at[pair, pl.ds(base, _K)], idx_v, sem)

    def wait(i, slot):
        chunk_v, idx_v, sem = bufs[slot]
        base = i * _K
        pltpu.make_async_copy(pred_hbm.at[pair, :, pl.ds(base, _K)],
                              chunk_v, sem).wait()
        pltpu.make_async_copy(idx_hbm.at[pair, pl.ds(base, _K)],
                              idx_v, sem).wait()

    def compute(slot):
        chunk_v, idx_v, _ = bufs[slot]

        def group_body(g, _):
            gb = g * _L
            rows = idx_v[pl.ds(gb, _L)]
            es = [jnp.exp(chunk_v[c, pl.ds(gb, _L)])
                  for c in range(c_classes)]
            t = list(es)
            while len(t) > 1:
                t = [t[i] + t[i + 1] for i in range(0, len(t) - 1, 2)] + (
                    [t[-1]] if len(t) % 2 else [])
            r = 1.0 / t[0]
            for c in range(c_classes):
                plsc.addupdate_scatter(
                    acc_v.at[pl.ds(c * n_points, n_points)], [rows],
                    es[c] * r)
            return _

        lax.fori_loop(0, _K // _L, group_body, None, unroll=3)

    start(0, 0)

    def chunk_pair(j, _):
        i = j * 2
        wait(i, 0)
        start(i + 1, 1)
        compute(0)
        wait(i + 1, 1)
        start(i + 2, 0)
        compute(1)
        return _

    lax.fori_loop(0, (n_chunks - 1) // 2, chunk_pair, None)
    wait(n_chunks - 1, 0)
    compute(0)


def _scatter_a_body(n_points, n_chunks, pred_hbm, idx_hbm, part_hbm,
                    acc_v, chunk0_v, chunk1_v, idx0_v, idx1_v, sem0, sem1):
    c_classes = chunk0_v.shape[0]
    pair = lax.axis_index("c") * 16 + lax.axis_index("s")

    zeros = jnp.zeros((_L,), jnp.float32)

    def zero_body(j, _):
        acc_v[pl.ds(j * _L, _L)] = zeros
        return _

    lax.fori_loop(0, n_points * c_classes // _L, zero_body, None)
    _scatter_chunks(n_points, n_chunks, pred_hbm, idx_hbm, pair,
                    acc_v, chunk0_v, chunk1_v, idx0_v, idx1_v, sem0, sem1)
    pltpu.sync_copy(acc_v, part_hbm.at[pair])


def _scatter_b_body(n_points, n_chunks, n_views, rows_per_sub,
                    pred_hbm, idx_hbm, w_hbm, parta_hbm,
                    part_hbm, out_hbm,
                    acc_v, chunk0_v, chunk1_v, idx0_v, idx1_v,
                    wbuf_v, obuf_v, sem0, sem1):
    c_classes = chunk0_v.shape[0]
    cc = lax.axis_index("c")
    sid = lax.axis_index("s")
    pair = cc * 16 + sid

    pltpu.sync_copy(parta_hbm.at[pair], acc_v)
    _scatter_chunks(n_points, n_chunks, pred_hbm, idx_hbm, pair,
                    acc_v, chunk0_v, chunk1_v, idx0_v, idx1_v, sem0, sem1)

    pltpu.sync_copy(acc_v, part_hbm.at[pair])
    plsc.subcore_barrier()

    lb = sid // n_views
    b = cc * 2 + lb
    nbase = (sid % n_views) * rows_per_sub

    def fcopy(v, c):
        return pltpu.make_async_copy(
            part_hbm.at[b * n_views + v, pl.ds(c * n_points + nbase,
                                               rows_per_sub)],
            acc_v.at[pl.ds((v * c_classes + c) * rows_per_sub,
                           rows_per_sub)],
            sem0)

    for v in range(n_views):
        for c in range(c_classes):
            fcopy(v, c).start()
    pltpu.sync_copy(w_hbm.at[pl.ds(b * n_views, n_views), :], wbuf_v)
    for v in range(n_views):
        for c in range(c_classes):
            fcopy(v, c).wait()

    zeros = jnp.zeros((_L,), jnp.float32)
    one = jnp.ones((_L,), jnp.float32)
    lane = lax.iota(jnp.int32, _L)

    def group_body(i, _):
        ib = i * _L
        acc = [zeros] * c_classes
        nvis = zeros
        for v in range(n_views):
            rows = [acc_v[pl.ds((v * c_classes + c) * rows_per_sub + ib, _L)]
                    for c in range(c_classes)]
            cnt = rows[0]
            for c in range(1, c_classes):
                cnt = cnt + rows[c]
            visv = jnp.where(cnt > 0.5, one, zeros)
            scale = (wbuf_v[v, :] * visv) / jnp.maximum(cnt, one)
            for c in range(c_classes):
                acc[c] = acc[c] + rows[c] * scale
            nvis = nvis + visv
        inv = one / jnp.maximum(nvis, one)
        locs = (ib + lane) * c_classes
        for c in range(c_classes):
            plsc.store_scatter(obuf_v, [locs + c], acc[c] * inv)
        return _

    lax.fori_loop(0, rows_per_sub // _L, group_body, None)
    pltpu.sync_copy(
        obuf_v, out_hbm.at[b, pl.ds(nbase * c_classes,
                                    rows_per_sub * c_classes)])


def kernel(points, predictions_2d, rendered_pix_to_point, views_weights,
           cls, parts_nb):
    b, n, _ = points.shape
    _, v, c, h, w = predictions_2d.shape
    hh = h // 2
    p2 = hh * w

    predA = predictions_2d[:, :, :, :hh, :].reshape(b * v, c, p2)
    idxA = rendered_pix_to_point[:, :, :hh, :].reshape(b * v, p2)
    predB = predictions_2d[:, :, :, hh:, :].reshape(b * v, c, p2)
    idxB = rendered_pix_to_point[:, :, hh:, :].reshape(b * v, p2)
    wt = jnp.broadcast_to(views_weights.reshape(b * v, 1),
                          (b * v, _L)).astype(jnp.float32)

    mesh = plsc.VectorSubcoreMesh(core_axis_name="c", subcore_axis_name="s")
    rows_per_sub = (b * n) // 32
    n_chunks = p2 // _K

    scatter_scratch = [
        pltpu.VMEM((n * c,), jnp.float32),
        pltpu.VMEM((c, _K), jnp.float32),
        pltpu.VMEM((c, _K), jnp.float32),
        pltpu.VMEM((_K,), jnp.int32),
        pltpu.VMEM((_K,), jnp.int32),
    ]
    sems = [pltpu.SemaphoreType.DMA, pltpu.SemaphoreType.DMA]

    parta = pl.kernel(
        functools.partial(_scatter_a_body, n, n_chunks),
        out_type=jax.ShapeDtypeStruct((b * v, n * c), jnp.float32),
        mesh=mesh,
        scratch_types=scatter_scratch + sems,
        compiler_params=pltpu.CompilerParams(needs_layout_passes=False),
        name="mvlift_scatter_a",
    )(predA, idxA)

    _, out = pl.kernel(
        functools.partial(_scatter_b_body, n, n_chunks, v, rows_per_sub),
        out_type=(jax.ShapeDtypeStruct((b * v, n * c), jnp.float32),
                  jax.ShapeDtypeStruct((b, n * c), jnp.float32)),
        mesh=mesh,
        scratch_types=scatter_scratch + [
            pltpu.VMEM((v, _L), jnp.float32),
            pltpu.VMEM((rows_per_sub * c,), jnp.float32),
        ] + sems,
        compiler_params=pltpu.CompilerParams(needs_layout_passes=False),
        name="mvlift_scatter_b",
    )(predB, idxB, wt, parta)
    return out.reshape(b, n, c)


# speedup vs baseline: 1.1051x; 1.1051x over previous
import functools

import jax
import jax.numpy as jnp
from jax import lax
from jax.experimental import pallas as pl
from jax.experimental.pallas import tpu as pltpu
from jax.experimental.pallas import tpu_sc as plsc

_L = 16
_K = 1024


def _fused_body(n_points, n_chunks, n_views, rows_per_sub,
                pred_hbm, idx_hbm, w_hbm, part_hbm, out_hbm,
                acc_v, chunk0_v, chunk1_v, idx0_v, idx1_v,
                wbuf_v, obuf_v, sem0, sem1):
    c_classes = chunk0_v.shape[0]
    cc = lax.axis_index("c")
    sid = lax.axis_index("s")
    pair = cc * 16 + sid

    zeros = jnp.zeros((_L,), jnp.float32)

    def zero_body(j, _):
        acc_v[pl.ds(j * _L, _L)] = zeros
        return _

    lax.fori_loop(0, n_points * c_classes // _L, zero_body, None)

    bufs = ((chunk0_v, idx0_v, sem0), (chunk1_v, idx1_v, sem1))

    def start(i, slot):
        chunk_v, idx_v, sem = bufs[slot]
        base = i * _K
        pltpu.async_copy(pred_hbm.at[pair, :, pl.ds(base, _K)], chunk_v, sem)
        pltpu.async_copy(idx_hbm.at[pair, pl.ds(base, _K)], idx_v, sem)

    def wait(i, slot):
        chunk_v, idx_v, sem = bufs[slot]
        base = i * _K
        pltpu.make_async_copy(pred_hbm.at[pair, :, pl.ds(base, _K)],
                              chunk_v, sem).wait()
        pltpu.make_async_copy(idx_hbm.at[pair, pl.ds(base, _K)],
                              idx_v, sem).wait()

    def compute(slot):
        chunk_v, idx_v, _ = bufs[slot]

        def group_body(g, _):
            gb = g * _L
            rows = idx_v[pl.ds(gb, _L)]
            es = [jnp.exp(chunk_v[c, pl.ds(gb, _L)])
                  for c in range(c_classes)]
            t = list(es)
            while len(t) > 1:
                t = [t[i] + t[i + 1] for i in range(0, len(t) - 1, 2)] + (
                    [t[-1]] if len(t) % 2 else [])
            r = 1.0 / t[0]
            for c in range(c_classes):
                plsc.addupdate_scatter(
                    acc_v.at[pl.ds(c * n_points, n_points)], [rows],
                    es[c] * r)
            return _

        lax.fori_loop(0, _K // _L, group_body, None, unroll=3)

    start(0, 0)

    def chunk_pair(j, _):
        i = j * 2
        wait(i, 0)
        start(i + 1, 1)
        compute(0)
        wait(i + 1, 1)
        start(i + 2, 0)
        compute(1)
        return _

    lax.fori_loop(0, (n_chunks - 1) // 2, chunk_pair, None)
    wait(n_chunks - 1, 0)
    compute(0)

    pltpu.sync_copy(acc_v, part_hbm.at[pair])
    plsc.subcore_barrier()

    lb = sid // n_views
    b = cc * 2 + lb
    nbase = (sid % n_views) * rows_per_sub

    def fcopy(v, c):
        return pltpu.make_async_copy(
            part_hbm.at[b * n_views + v, pl.ds(c * n_points + nbase,
                                               rows_per_sub)],
            acc_v.at[pl.ds((v * c_classes + c) * rows_per_sub,
                           rows_per_sub)],
            sem0)

    for v in range(n_views):
        for c in range(c_classes):
            fcopy(v, c).start()
    pltpu.sync_copy(w_hbm.at[pl.ds(b * n_views, n_views), :], wbuf_v)
    for v in range(n_views):
        for c in range(c_classes):
            fcopy(v, c).wait()

    one = jnp.ones((_L,), jnp.float32)
    lane = lax.iota(jnp.int32, _L)

    def group_body(i, _):
        ib = i * _L
        acc = [zeros] * c_classes
        nvis = zeros
        for v in range(n_views):
            rows = [acc_v[pl.ds((v * c_classes + c) * rows_per_sub + ib, _L)]
                    for c in range(c_classes)]
            cnt = rows[0]
            for c in range(1, c_classes):
                cnt = cnt + rows[c]
            visv = jnp.where(cnt > 0.5, one, zeros)
            scale = (wbuf_v[v, :] * visv) / jnp.maximum(cnt, one)
            for c in range(c_classes):
                acc[c] = acc[c] + rows[c] * scale
            nvis = nvis + visv
        inv = one / jnp.maximum(nvis, one)
        locs = (ib + lane) * c_classes
        for c in range(c_classes):
            plsc.store_scatter(obuf_v, [locs + c], acc[c] * inv)
        return _

    lax.fori_loop(0, rows_per_sub // _L, group_body, None)
    pltpu.sync_copy(
        obuf_v, out_hbm.at[b, pl.ds(nbase * c_classes,
                                    rows_per_sub * c_classes)])


def kernel(points, predictions_2d, rendered_pix_to_point, views_weights,
           cls, parts_nb):
    b, n, _ = points.shape
    _, v, c, h, w = predictions_2d.shape
    p = h * w

    pred = predictions_2d.reshape(b * v, c, p)
    idx = rendered_pix_to_point.reshape(b * v, p)
    wt = jnp.broadcast_to(views_weights.reshape(b * v, 1),
                          (b * v, _L)).astype(jnp.float32)

    mesh = plsc.VectorSubcoreMesh(core_axis_name="c", subcore_axis_name="s")
    rows_per_sub = (b * n) // 32

    _, out = pl.kernel(
        functools.partial(_fused_body, n, p // _K, v, rows_per_sub),
        out_type=(jax.ShapeDtypeStruct((b * v, n * c), jnp.float32),
                  jax.ShapeDtypeStruct((b, n * c), jnp.float32)),
        mesh=mesh,
        scratch_types=[
            pltpu.VMEM((n * c,), jnp.float32),
            pltpu.VMEM((c, _K), jnp.float32),
            pltpu.VMEM((c, _K), jnp.float32),
            pltpu.VMEM((_K,), jnp.int32),
            pltpu.VMEM((_K,), jnp.int32),
            pltpu.VMEM((v, _L), jnp.float32),
            pltpu.VMEM((rows_per_sub * c,), jnp.float32),
            pltpu.SemaphoreType.DMA,
            pltpu.SemaphoreType.DMA,
        ],
        compiler_params=pltpu.CompilerParams(needs_layout_passes=False),
        name="mvlift_fused",
    )(pred, idx, wt)
    return out.reshape(b, n, c)
